# baseline (speedup 1.0000x reference)
import functools

import jax
import jax.numpy as jnp
from jax import lax
from jax.experimental import pallas as pl
from jax.experimental.pallas import tpu as pltpu
from jax.experimental.pallas import tpu_sc as plsc

N = 10000
E = 320000
D = 128
F = 64
G = 500
GP = 512
P = 4096
C = 8
NPAD = 10240

NC = 2
NS = 16
NW = NC * NS
EPT = E // NW
CH = 125
NCH = EPT // CH
DEG_ROWS = NPAD // 128

_HI = jax.lax.Precision.HIGHEST


def _dot(a, b):
    return jax.lax.dot_general(a, b, (((a.ndim - 1,), (0,)), ((), ())),
                               precision=_HI, preferred_element_type=jnp.float32)


def _sc_degree(dst_r, ones_col, zcol):
    mesh = plsc.VectorSubcoreMesh(core_axis_name="c", subcore_axis_name="s")

    @functools.partial(
        pl.kernel,
        out_type=jax.ShapeDtypeStruct((NC, NPAD, 16), jnp.float32),
        mesh=mesh,
        scratch_types=[
            pltpu.VMEM((NCH, CH), jnp.int32),
            pltpu.VMEM((CH, 16), jnp.float32),
            pltpu.VMEM_SHARED((NPAD, 16), jnp.float32),
        ],
        compiler_params=pltpu.CompilerParams(use_tc_tiling_on_sc=False),
    )
    def k(dst_hbm, ones_hbm, z_hbm, out_hbm, dstv, onesv, sdeg):
        ci = lax.axis_index("c")
        si = lax.axis_index("s")
        wid = si * NC + ci
        pltpu.sync_copy(dst_hbm.at[wid], dstv)
        pltpu.sync_copy(ones_hbm, onesv)
        pltpu.sync_copy(z_hbm, sdeg.at[pl.ds(si * (NPAD // NS), NPAD // NS)])
        plsc.subcore_barrier()

        def body(j, _):
            pltpu.sync_copy(onesv, sdeg.at[dstv.at[j]], add=True)
            return _

        lax.fori_loop(0, NCH, body, 0)
        plsc.subcore_barrier()
        pltpu.sync_copy(sdeg.at[pl.ds(si * (NPAD // NS), NPAD // NS)],
                        out_hbm.at[ci, pl.ds(si * (NPAD // NS), NPAD // NS)])

    return k(dst_r, ones_col, zcol)


def _sc_edge_scatter(mp, src_r, dst_r, zrows):
    mesh = plsc.VectorSubcoreMesh(core_axis_name="c", subcore_axis_name="s")

    @functools.partial(
        pl.kernel,
        out_type=jax.ShapeDtypeStruct((NC, NPAD, F), jnp.float32),
        mesh=mesh,
        scratch_types=[
            pltpu.VMEM((NCH, CH), jnp.int32),
            pltpu.VMEM((NCH, CH), jnp.int32),
            pltpu.VMEM((CH, F), jnp.float32),
            pltpu.VMEM((CH, F), jnp.float32),
            pltpu.VMEM_SHARED((NPAD, F), jnp.float32),
            pltpu.SemaphoreType.DMA,
            pltpu.SemaphoreType.DMA,
        ],
        compiler_params=pltpu.CompilerParams(use_tc_tiling_on_sc=False),
    )
    def k(mp_hbm, src_hbm, dst_hbm, z_hbm, out_hbm,
          srcv, dstv, rows0, rows1, acc, sem0, sem1):
        ci = lax.axis_index("c")
        si = lax.axis_index("s")
        wid = si * NC + ci
        pltpu.sync_copy(src_hbm.at[wid], srcv)
        pltpu.sync_copy(dst_hbm.at[wid], dstv)
        pltpu.sync_copy(z_hbm, acc.at[pl.ds(si * (NPAD // NS), NPAD // NS)])
        plsc.subcore_barrier()

        pltpu.async_copy(mp_hbm.at[srcv.at[0]], rows0, sem0)
        pltpu.async_copy(mp_hbm.at[srcv.at[1]], rows1, sem1)

        def body(g, carry):
            j = g * 2
            pltpu.make_async_copy(mp_hbm.at[srcv.at[j]], rows0, sem0).wait()
            pltpu.sync_copy(rows0, acc.at[dstv.at[j]], add=True)

            @pl.when(j + 2 < NCH)
            def _():
                pltpu.async_copy(mp_hbm.at[srcv.at[j + 2]], rows0, sem0)

            pltpu.make_async_copy(mp_hbm.at[srcv.at[j + 1]], rows1, sem1).wait()
            pltpu.sync_copy(rows1, acc.at[dstv.at[j + 1]], add=True)

            @pl.when(j + 3 < NCH)
            def _():
                pltpu.async_copy(mp_hbm.at[srcv.at[j + 3]], rows1, sem1)

            return carry

        lax.fori_loop(0, NCH // 2, body, 0)
        plsc.subcore_barrier()
        pltpu.sync_copy(acc.at[pl.ds(si * (NPAD // NS), NPAD // NS)],
                        out_hbm.at[ci, pl.ds(si * (NPAD // NS), NPAD // NS)])

    return k(mp, src_r, dst_r, zrows)


def _tc_first_layer(x, w1c, degp):
    def k(x_ref, w_ref, degp_ref, dinv_ref, mp_ref):
        deg = 1.0 + degp_ref[0] + degp_ref[1]
        dinv = lax.rsqrt(deg)
        dinv_ref[...] = dinv
        mp_ref[...] = dinv * _dot(x_ref[...], w_ref[...])

    return pl.pallas_call(
        k,
        out_shape=(jax.ShapeDtypeStruct((N, 1), jnp.float32),
                   jax.ShapeDtypeStruct((N, F), jnp.float32)),
    )(x, w1c, degp)


def _tc_second_layer(acc, mp1, dinv, b1, w2c):
    def k(acc_ref, mp_ref, dinv_ref, b_ref, w_ref, out_ref):
        dinv = dinv_ref[...]
        h1 = jax.nn.relu(dinv * (acc_ref[0] + acc_ref[1] + mp_ref[...])
                         + b_ref[...])
        out_ref[...] = dinv * _dot(h1, w_ref[...])

    return pl.pallas_call(
        k,
        out_shape=jax.ShapeDtypeStruct((N, F), jnp.float32),
    )(acc, mp1, dinv, b1.reshape(1, F), w2c)


def _tc_tail(acc, mp2, dinv, b2, batch3, ddflat, ddT, eclf, w1, b1, w2, b2e):
    NB = 10
    CHN = N // NB

    def k(acc_ref, mp_ref, dinv_ref, b_ref, batch_ref, ddflat_ref, ddT_ref,
          ecl_ref, w1_ref, b1_ref, w2_ref, b2_ref, sums_ref, cnts_ref, out_ref):
        i = pl.program_id(0)

        @pl.when(i == 0)
        def _():
            sums_ref[...] = jnp.zeros_like(sums_ref)
            cnts_ref[...] = jnp.zeros_like(cnts_ref)

        dinv = dinv_ref[...]
        h2 = jax.nn.relu(dinv * (acc_ref[0] + acc_ref[1] + mp_ref[...])
                         + b_ref[...])
        b = batch_ref[0]
        gids = lax.broadcasted_iota(jnp.int32, (GP, CHN), 0).astype(jnp.float32)
        oh = (gids == b).astype(jnp.float32)
        sums_ref[...] += _dot(oh, h2)
        cnts_ref[...] += jnp.sum(oh, axis=1, keepdims=True)

        @pl.when(i == NB - 1)
        def _():
            ge = sums_ref[...] / jnp.maximum(cnts_ref[...], 1.0)
            gid_col = lax.broadcasted_iota(jnp.int32, (GP, 1024), 0).astype(jnp.float32)
            pres = jnp.zeros((GP, 1), jnp.float32)
            for sblk in range(2 * P // 1024):
                blk = ddflat_ref[0, pl.ds(sblk * 1024, 1024)].reshape(1, 1024)
                pres = jnp.maximum(
                    pres, jnp.max((gid_col == blk).astype(jnp.float32),
                                  axis=1, keepdims=True))
            r_i = lax.broadcasted_iota(jnp.int32, (GP, GP), 0).astype(jnp.float32)
            c_i = lax.broadcasted_iota(jnp.int32, (GP, GP), 1).astype(jnp.float32)
            tri = (c_i < r_i).astype(jnp.float32)
            ranks = _dot(tri, pres)

            iota_l = lax.broadcasted_iota(jnp.int32, (P, GP), 1).astype(jnp.float32)
            d0 = ddT_ref[:, 0:1]
            d1 = ddT_ref[:, 1:2]
            r0 = _dot((iota_l == d0).astype(jnp.float32), ranks)
            r1 = _dot((iota_l == d1).astype(jnp.float32), ranks)
            from_e = _dot((iota_l == r0).astype(jnp.float32), ge)
            to_e = _dot((iota_l == r1).astype(jnp.float32), ge)
            pairs = jnp.concatenate([from_e, to_e], axis=1)

            ecl = ecl_ref[...]
            preds = jnp.zeros((P, 1), jnp.float32)
            for c in range(C):
                h = jax.nn.relu(_dot(pairs, w1_ref[c]) + b1_ref[c:c + 1, :])
                p = _dot(h, w2_ref[c]) + b2_ref[c:c + 1, :]
                preds = preds + (ecl == float(c)).astype(jnp.float32) * p
            out_ref[...] = preds

    return pl.pallas_call(
        k,
        grid=(NB,),
        in_specs=[
            pl.BlockSpec((NC, CHN, F), lambda i: (0, i, 0)),
            pl.BlockSpec((CHN, F), lambda i: (i, 0)),
            pl.BlockSpec((CHN, 1), lambda i: (i, 0)),
            pl.BlockSpec((1, F), lambda i: (0, 0)),
            pl.BlockSpec((1, 1, CHN), lambda i: (i, 0, 0)),
            pl.BlockSpec((1, 2 * P), lambda i: (0, 0)),
            pl.BlockSpec((P, 2), lambda i: (0, 0)),
            pl.BlockSpec((P, 1), lambda i: (0, 0)),
            pl.BlockSpec((C, 2 * F, F), lambda i: (0, 0, 0)),
            pl.BlockSpec((C, F), lambda i: (0, 0)),
            pl.BlockSpec((C, F, 1), lambda i: (0, 0, 0)),
            pl.BlockSpec((C, 1), lambda i: (0, 0)),
        ],
        out_specs=(pl.BlockSpec((GP, F), lambda i: (0, 0)),
                   pl.BlockSpec((GP, 1), lambda i: (0, 0)),
                   pl.BlockSpec((P, 1), lambda i: (0, 0))),
        out_shape=(jax.ShapeDtypeStruct((GP, F), jnp.float32),
                   jax.ShapeDtypeStruct((GP, 1), jnp.float32),
                   jax.ShapeDtypeStruct((P, 1), jnp.float32)),
    )(acc, mp2, dinv, b2.reshape(1, F), batch3, ddflat, ddT, eclf,
      w1, b1, w2, b2e)[2]


def kernel(x, edge_index, batch, drug_drug_batch, edge_cell_lines,
           conv1_w, conv1_b, conv2_w, conv2_b, w1, b1, w2, b2):
    src = edge_index[0]
    dst = edge_index[1]
    src_r = src.reshape(NW, NCH, CH)
    dst_r = dst.reshape(NW, NCH, CH)

    ones_col = jnp.zeros((CH, 16), jnp.float32).at[:, 0].set(1.0)
    zcol = jnp.zeros((NPAD // NS, 16), jnp.float32)
    zrows = jnp.zeros((NPAD // NS, F), jnp.float32)

    degp = _sc_degree(dst_r, ones_col, zcol)
    degp_col = degp[:, :N, 0:1]

    dinv, mp1 = _tc_first_layer(x, conv1_w, degp_col)
    acc1 = _sc_edge_scatter(mp1, src_r, dst_r, zrows)[:, :N]
    mp2 = _tc_second_layer(acc1, mp1, dinv, conv1_b, conv2_w)
    acc2 = _sc_edge_scatter(mp2, src_r, dst_r, zrows)[:, :N]

    batch3 = batch.astype(jnp.float32).reshape(10, 1, N // 10)
    ddf = drug_drug_batch.astype(jnp.float32)
    ddflat = ddf.reshape(1, 2 * P)
    ddT = ddf.T
    eclf = edge_cell_lines.astype(jnp.float32).reshape(P, 1)
    preds = _tc_tail(acc2, mp2, dinv, conv2_b, batch3, ddflat, ddT, eclf,
                     w1, b1, w2, b2)
    return preds


# speedup vs baseline: 1.1187x; 1.1187x over previous
import functools

import jax
import jax.numpy as jnp
from jax import lax
from jax.experimental import pallas as pl
from jax.experimental.pallas import tpu as pltpu
from jax.experimental.pallas import tpu_sc as plsc

N = 10000
E = 320000
D = 128
F = 64
G = 500
GP = 512
P = 4096
C = 8
NPAD = 10240

NC = 2
NS = 16
NW = NC * NS
EPT = E // NW
CH = 125
NCH = EPT // CH
DEG_ROWS = NPAD // 128

_HI = jax.lax.Precision.HIGHEST


def _dot(a, b):
    return jax.lax.dot_general(a, b, (((a.ndim - 1,), (0,)), ((), ())),
                               precision=_HI, preferred_element_type=jnp.float32)


def _sc_degree(dst_r, ones_col, zcol):
    mesh = plsc.VectorSubcoreMesh(core_axis_name="c", subcore_axis_name="s")

    @functools.partial(
        pl.kernel,
        out_type=jax.ShapeDtypeStruct((NC, NPAD, 16), jnp.float32),
        mesh=mesh,
        scratch_types=[
            pltpu.VMEM((NCH, CH), jnp.int32),
            pltpu.VMEM((CH, 16), jnp.float32),
            pltpu.VMEM_SHARED((NPAD, 16), jnp.float32),
        ],
        compiler_params=pltpu.CompilerParams(use_tc_tiling_on_sc=False),
    )
    def k(dst_hbm, ones_hbm, z_hbm, out_hbm, dstv, onesv, sdeg):
        ci = lax.axis_index("c")
        si = lax.axis_index("s")
        wid = si * NC + ci
        pltpu.sync_copy(dst_hbm.at[wid], dstv)
        pltpu.sync_copy(ones_hbm, onesv)
        pltpu.sync_copy(z_hbm, sdeg.at[pl.ds(si * (NPAD // NS), NPAD // NS)])
        plsc.subcore_barrier()

        def body(j, _):
            pltpu.sync_copy(onesv, sdeg.at[dstv.at[j]], add=True)
            return _

        lax.fori_loop(0, NCH, body, 0)
        plsc.subcore_barrier()
        pltpu.sync_copy(sdeg.at[pl.ds(si * (NPAD // NS), NPAD // NS)],
                        out_hbm.at[ci, pl.ds(si * (NPAD // NS), NPAD // NS)])

    return k(dst_r, ones_col, zcol)


def _sc_edge_scatter(mp, src_r, dst_r, zrows):
    mesh = plsc.VectorSubcoreMesh(core_axis_name="c", subcore_axis_name="s")

    @functools.partial(
        pl.kernel,
        out_type=jax.ShapeDtypeStruct((NC, NPAD, F), jnp.float32),
        mesh=mesh,
        scratch_types=[
            pltpu.VMEM((NCH, CH), jnp.int32),
            pltpu.VMEM((NCH, CH), jnp.int32),
            pltpu.VMEM((CH, F), jnp.float32),
            pltpu.VMEM((CH, F), jnp.float32),
            pltpu.VMEM_SHARED((NPAD, F), jnp.float32),
            pltpu.SemaphoreType.DMA,
            pltpu.SemaphoreType.DMA,
        ],
        compiler_params=pltpu.CompilerParams(use_tc_tiling_on_sc=False),
    )
    def k(mp_hbm, src_hbm, dst_hbm, z_hbm, out_hbm,
          srcv, dstv, rows0, rows1, acc, sem0, sem1):
        ci = lax.axis_index("c")
        si = lax.axis_index("s")
        wid = si * NC + ci
        pltpu.sync_copy(src_hbm.at[wid], srcv)
        pltpu.sync_copy(dst_hbm.at[wid], dstv)
        pltpu.sync_copy(z_hbm, acc.at[pl.ds(si * (NPAD // NS), NPAD // NS)])
        plsc.subcore_barrier()

        pltpu.async_copy(mp_hbm.at[srcv.at[0]], rows0, sem0)
        pltpu.async_copy(mp_hbm.at[srcv.at[1]], rows1, sem1)

        def body(g, carry):
            j = g * 2
            pltpu.make_async_copy(mp_hbm.at[srcv.at[j]], rows0, sem0).wait()
            pltpu.sync_copy(rows0, acc.at[dstv.at[j]], add=True)

            @pl.when(j + 2 < NCH)
            def _():
                pltpu.async_copy(mp_hbm.at[srcv.at[j + 2]], rows0, sem0)

            pltpu.make_async_copy(mp_hbm.at[srcv.at[j + 1]], rows1, sem1).wait()
            pltpu.sync_copy(rows1, acc.at[dstv.at[j + 1]], add=True)

            @pl.when(j + 3 < NCH)
            def _():
                pltpu.async_copy(mp_hbm.at[srcv.at[j + 3]], rows1, sem1)

            return carry

        lax.fori_loop(0, NCH // 2, body, 0)
        plsc.subcore_barrier()
        pltpu.sync_copy(acc.at[pl.ds(si * (NPAD // NS), NPAD // NS)],
                        out_hbm.at[ci, pl.ds(si * (NPAD // NS), NPAD // NS)])

    return k(mp, src_r, dst_r, zrows)


def _tc_first_layer(x, w1c, degp):
    def k(x_ref, w_ref, degp_ref, dinv_ref, mp_ref):
        deg = 1.0 + degp_ref[0] + degp_ref[1]
        dinv = lax.rsqrt(deg)
        dinv_ref[...] = dinv
        mp_ref[...] = dinv * _dot(x_ref[...], w_ref[...])

    return pl.pallas_call(
        k,
        out_shape=(jax.ShapeDtypeStruct((N, 1), jnp.float32),
                   jax.ShapeDtypeStruct((N, F), jnp.float32)),
    )(x, w1c, degp)


def _tc_second_layer(acc, mp1, dinv, b1, w2c):
    def k(acc_ref, mp_ref, dinv_ref, b_ref, w_ref, out_ref):
        dinv = dinv_ref[...]
        h1 = jax.nn.relu(dinv * (acc_ref[0] + acc_ref[1] + mp_ref[...])
                         + b_ref[...])
        out_ref[...] = dinv * _dot(h1, w_ref[...])

    return pl.pallas_call(
        k,
        out_shape=jax.ShapeDtypeStruct((N, F), jnp.float32),
    )(acc, mp1, dinv, b1.reshape(1, F), w2c)


def _tc_graph_embeds(acc, mp2, dinv, b2, batch3):
    NB = 10
    CHN = N // NB

    def k(acc_ref, mp_ref, dinv_ref, b_ref, batch_ref, sums_ref, cnts_ref):
        i = pl.program_id(0)

        @pl.when(i == 0)
        def _():
            sums_ref[...] = jnp.zeros_like(sums_ref)
            cnts_ref[...] = jnp.zeros_like(cnts_ref)

        dinv = dinv_ref[...]
        h2 = jax.nn.relu(dinv * (acc_ref[0] + acc_ref[1] + mp_ref[...])
                         + b_ref[...])
        b = batch_ref[0]
        gids = lax.broadcasted_iota(jnp.int32, (GP, CHN), 0).astype(jnp.float32)
        oh = (gids == b).astype(jnp.float32)
        sums_ref[...] += _dot(oh, h2)
        cnts_ref[...] += jnp.sum(oh, axis=1, keepdims=True)

    return pl.pallas_call(
        k,
        grid=(NB,),
        in_specs=[
            pl.BlockSpec((NC, CHN, F), lambda i: (0, i, 0)),
            pl.BlockSpec((CHN, F), lambda i: (i, 0)),
            pl.BlockSpec((CHN, 1), lambda i: (i, 0)),
            pl.BlockSpec((1, F), lambda i: (0, 0)),
            pl.BlockSpec((1, 1, CHN), lambda i: (i, 0, 0)),
        ],
        out_specs=(pl.BlockSpec((GP, F), lambda i: (0, 0)),
                   pl.BlockSpec((GP, 1), lambda i: (0, 0))),
        out_shape=(jax.ShapeDtypeStruct((GP, F), jnp.float32),
                   jax.ShapeDtypeStruct((GP, 1), jnp.float32)),
    )(acc, mp2, dinv, b2.reshape(1, F), batch3)


def _tc_pairs_experts(sums, cnts, ddflat, ddT, eclf, w1, b1, w2, b2):
    def k(sums_ref, cnts_ref, ddflat_ref, ddT_ref, ecl_ref,
          w1_ref, b1_ref, w2_ref, b2_ref, out_ref):
        ge = sums_ref[...] / jnp.maximum(cnts_ref[...], 1.0)
        gid_col = lax.broadcasted_iota(jnp.int32, (GP, 1024), 0).astype(jnp.float32)
        pres = jnp.zeros((GP, 1), jnp.float32)
        for sblk in range(2 * P // 1024):
            blk = ddflat_ref[0, pl.ds(sblk * 1024, 1024)].reshape(1, 1024)
            pres = jnp.maximum(
                pres, jnp.max((gid_col == blk).astype(jnp.float32),
                              axis=1, keepdims=True))
        r_i = lax.broadcasted_iota(jnp.int32, (GP, GP), 0).astype(jnp.float32)
        c_i = lax.broadcasted_iota(jnp.int32, (GP, GP), 1).astype(jnp.float32)
        tri = (c_i < r_i).astype(jnp.float32)
        ranks = _dot(tri, pres)

        iota_l = lax.broadcasted_iota(jnp.int32, (P, GP), 1).astype(jnp.float32)
        d0 = ddT_ref[:, 0:1]
        d1 = ddT_ref[:, 1:2]
        r0 = _dot((iota_l == d0).astype(jnp.float32), ranks)
        r1 = _dot((iota_l == d1).astype(jnp.float32), ranks)
        from_e = _dot((iota_l == r0).astype(jnp.float32), ge)
        to_e = _dot((iota_l == r1).astype(jnp.float32), ge)
        pairs = jnp.concatenate([from_e, to_e], axis=1)

        ecl = ecl_ref[...]
        preds = jnp.zeros((P, 1), jnp.float32)
        for c in range(C):
            h = jax.nn.relu(_dot(pairs, w1_ref[c]) + b1_ref[c:c + 1, :])
            p = _dot(h, w2_ref[c]) + b2_ref[c:c + 1, :]
            preds = preds + (ecl == float(c)).astype(jnp.float32) * p
        out_ref[...] = preds

    return pl.pallas_call(
        k,
        out_shape=jax.ShapeDtypeStruct((P, 1), jnp.float32),
    )(sums, cnts, ddflat, ddT, eclf, w1, b1, w2, b2)


def kernel(x, edge_index, batch, drug_drug_batch, edge_cell_lines,
           conv1_w, conv1_b, conv2_w, conv2_b, w1, b1, w2, b2):
    src = edge_index[0]
    dst = edge_index[1]
    src_r = src.reshape(NW, NCH, CH)
    dst_r = dst.reshape(NW, NCH, CH)

    ones_col = jnp.zeros((CH, 16), jnp.float32).at[:, 0].set(1.0)
    zcol = jnp.zeros((NPAD // NS, 16), jnp.float32)
    zrows = jnp.zeros((NPAD // NS, F), jnp.float32)

    degp = _sc_degree(dst_r, ones_col, zcol)
    degp_col = degp[:, :N, 0:1]

    dinv, mp1 = _tc_first_layer(x, conv1_w, degp_col)
    acc1 = _sc_edge_scatter(mp1, src_r, dst_r, zrows)[:, :N]
    mp2 = _tc_second_layer(acc1, mp1, dinv, conv1_b, conv2_w)
    acc2 = _sc_edge_scatter(mp2, src_r, dst_r, zrows)[:, :N]

    batch3 = batch.astype(jnp.float32).reshape(10, 1, N // 10)
    sums, cnts = _tc_graph_embeds(acc2, mp2, dinv, conv2_b, batch3)

    ddf = drug_drug_batch.astype(jnp.float32)
    ddflat = ddf.reshape(1, 2 * P)
    ddT = ddf.T
    eclf = edge_cell_lines.astype(jnp.float32).reshape(P, 1)
    preds = _tc_pairs_experts(sums, cnts, ddflat, ddT, eclf, w1, b1, w2, b2)
    return preds
